# Initial kernel scaffold; baseline (speedup 1.0000x reference)
import jax
import jax.numpy as jnp
from jax.experimental import pallas as pl


def kernel(Adjacency, node_features, edge_attributes, W, b):
    raise NotImplementedError("write your pallas kernel here")


# speedup vs baseline: 3.3044x; 3.3044x over previous
import functools

import jax
import jax.numpy as jnp
from jax import lax
from jax.experimental import pallas as pl
from jax.experimental.pallas import tpu as pltpu
from jax.experimental.pallas import tpu_sc as plsc

N_NODES = 10000
N_EDGES = 320000
D_FEAT = 128
D_EDGE = 16
D_OUT = 128

NC = 2
NS = 16
NW = NC * NS
EPW = N_EDGES // NW
C = 80
NCH = EPW // C
SPLIT = 632
TAIL = N_NODES - 15 * SPLIT


def _ab_body(x_ref, w1_ref, w2_ref, b_ref, a_ref, bo_ref):
    x = x_ref[...]
    w2 = w2_ref[...]
    wd = w1_ref[...] - w2
    a_ref[...] = jnp.dot(x, wd, preferred_element_type=jnp.float32) + b_ref[...]
    bo_ref[...] = jnp.dot(x, w2, preferred_element_type=jnp.float32)


def _compute_ab(x, w1, w2, b2):
    grid = 10
    rows = N_NODES // grid
    return pl.pallas_call(
        _ab_body,
        grid=(grid,),
        in_specs=[
            pl.BlockSpec((rows, D_FEAT), lambda i: (i, 0)),
            pl.BlockSpec((D_FEAT, D_OUT), lambda i: (0, 0)),
            pl.BlockSpec((D_FEAT, D_OUT), lambda i: (0, 0)),
            pl.BlockSpec((1, D_OUT), lambda i: (0, 0)),
        ],
        out_specs=[
            pl.BlockSpec((rows, D_OUT), lambda i: (i, 0)),
            pl.BlockSpec((rows, D_OUT), lambda i: (i, 0)),
        ],
        out_shape=[
            jax.ShapeDtypeStruct((N_NODES, D_OUT), jnp.float32),
            jax.ShapeDtypeStruct((N_NODES, D_OUT), jnp.float32),
        ],
    )(x, w1, w2, b2)


def _e_body(ea_ref, w3_ref, e_ref):
    e_ref[...] = jnp.dot(ea_ref[...], w3_ref[...],
                         preferred_element_type=jnp.float32)


def _compute_e(ea, w3):
    grid = 125
    rows = N_EDGES // grid
    return pl.pallas_call(
        _e_body,
        grid=(grid,),
        in_specs=[
            pl.BlockSpec((rows, D_EDGE), lambda i: (i, 0)),
            pl.BlockSpec((D_EDGE, D_OUT), lambda i: (0, 0)),
        ],
        out_specs=pl.BlockSpec((rows, D_OUT), lambda i: (i, 0)),
        out_shape=jax.ShapeDtypeStruct((N_EDGES, D_OUT), jnp.float32),
    )(ea, w3)


def _sc_body(a_hbm, b_hbm, e_hbm, dst_hbm, src_hbm, out_hbm,
             buf_a, buf_b, buf_e, dst_g, dst_s, src_g, acc,
             sem_a, sem_b, sem_e):
    c = lax.axis_index("c")
    s = lax.axis_index("s")
    wid = s * NC + c

    def zero_buf(e, carry):
        for j in range(8):
            buf_a[e, pl.ds(j * 16, 16)] = jnp.zeros((16,), jnp.float32)
        return carry
    lax.fori_loop(0, C, zero_buf, 0)
    row0 = s * SPLIT
    for k in range(TAIL // C):
        pltpu.sync_copy(buf_a, acc.at[pl.ds(row0 + k * C, C)])
    pltpu.sync_copy(buf_a.at[pl.ds(0, TAIL % C)],
                    acc.at[pl.ds(row0 + (TAIL // C) * C, TAIL % C)])

    @pl.when(s < NS - 1)
    def _zero_rest():
        pltpu.sync_copy(buf_a, acc.at[pl.ds(row0 + TAIL, C)])
        pltpu.sync_copy(buf_a.at[pl.ds(0, SPLIT - TAIL - C)],
                        acc.at[pl.ds(row0 + TAIL + C, SPLIT - TAIL - C)])

    plsc.subcore_barrier()

    base0 = wid * EPW

    def chunk(i, carry):
        base = base0 + i * C
        pltpu.sync_copy(dst_hbm.at[pl.ds(base, C)], dst_g)
        pltpu.sync_copy(dst_hbm.at[pl.ds(base, C)], dst_s.at[0])
        pltpu.sync_copy(src_hbm.at[pl.ds(base, C)], src_g)
        cp_a = pltpu.async_copy(a_hbm.at[dst_g], buf_a, sem_a)
        cp_b = pltpu.async_copy(b_hbm.at[src_g], buf_b, sem_b)
        cp_e = pltpu.async_copy(e_hbm.at[pl.ds(base, C)], buf_e, sem_e)
        cp_a.wait()
        cp_b.wait()
        cp_e.wait()

        def compute(e, inner):
            for j in range(8):
                sl = pl.ds(j * 16, 16)
                buf_a[e, sl] = jnp.maximum(
                    buf_a[e, sl] + buf_b[e, sl] + buf_e[e, sl], 0.0)
            return inner
        lax.fori_loop(0, C, compute, 0)

        pltpu.sync_copy(buf_a, acc.at[dst_s.at[0]], add=True)
        return carry

    lax.fori_loop(0, NCH, chunk, 0)
    plsc.subcore_barrier()
    pltpu.sync_copy(acc.at[pl.ds(row0, TAIL)],
                    out_hbm.at[c, pl.ds(row0, TAIL)])

    @pl.when(s < NS - 1)
    def _copy_rest():
        pltpu.sync_copy(acc.at[pl.ds(row0 + TAIL, SPLIT - TAIL)],
                        out_hbm.at[c, pl.ds(row0 + TAIL, SPLIT - TAIL)])


_sc_edge = functools.partial(
    pl.kernel,
    mesh=plsc.VectorSubcoreMesh(core_axis_name="c", subcore_axis_name="s",
                                num_cores=NC, num_subcores=NS),
    out_type=jax.ShapeDtypeStruct((NC, N_NODES, D_OUT), jnp.float32),
    scratch_types=[
        pltpu.VMEM((C, D_OUT), jnp.float32),
        pltpu.VMEM((C, D_OUT), jnp.float32),
        pltpu.VMEM((C, D_OUT), jnp.float32),
        pltpu.VMEM((C,), jnp.int32),
        pltpu.VMEM((1, C), jnp.int32),
        pltpu.VMEM((C,), jnp.int32),
        pltpu.VMEM_SHARED((N_NODES, D_OUT), jnp.float32),
        pltpu.SemaphoreType.DMA,
        pltpu.SemaphoreType.DMA,
        pltpu.SemaphoreType.DMA,
    ],
)(_sc_body)


def _add_body(p_ref, o_ref):
    o_ref[...] = p_ref[0] + p_ref[1]


def _add_partials(p):
    grid = 10
    rows = N_NODES // grid
    return pl.pallas_call(
        _add_body,
        grid=(grid,),
        in_specs=[pl.BlockSpec((NC, rows, D_OUT), lambda i: (0, i, 0))],
        out_specs=pl.BlockSpec((rows, D_OUT), lambda i: (i, 0)),
        out_shape=jax.ShapeDtypeStruct((N_NODES, D_OUT), jnp.float32),
    )(p)


def kernel(Adjacency, node_features, edge_attributes, W, b):
    src = Adjacency[0].astype(jnp.int32)
    dst = Adjacency[1].astype(jnp.int32)
    w1 = W[:D_FEAT]
    w2 = W[D_FEAT:2 * D_FEAT]
    w3 = W[2 * D_FEAT:]
    b2 = b.reshape(1, D_OUT)
    a_tab, b_tab = _compute_ab(node_features, w1, w2, b2)
    e_tab = _compute_e(edge_attributes, w3)
    partials = _sc_edge(a_tab, b_tab, e_tab, dst, src)
    return _add_partials(partials)


# speedup vs baseline: 5.1755x; 1.5663x over previous
import functools

import jax
import jax.numpy as jnp
from jax import lax
from jax.experimental import pallas as pl
from jax.experimental.pallas import tpu as pltpu
from jax.experimental.pallas import tpu_sc as plsc

N_NODES = 10000
N_EDGES = 320000
D_FEAT = 128
D_EDGE = 16
D_OUT = 128

NC = 2
NS = 16
NW = NC * NS
EPW = N_EDGES // NW
C = 40
NCH = EPW // C
SPLIT = 632
TAIL = N_NODES - 15 * SPLIT


def _ab_body(x_ref, w1_ref, w2_ref, b_ref, a_ref, bo_ref):
    x = x_ref[...]
    w2 = w2_ref[...]
    wd = w1_ref[...] - w2
    a_ref[...] = jnp.dot(x, wd, preferred_element_type=jnp.float32) + b_ref[...]
    bo_ref[...] = jnp.dot(x, w2, preferred_element_type=jnp.float32)


def _compute_ab(x, w1, w2, b2):
    grid = 10
    rows = N_NODES // grid
    return pl.pallas_call(
        _ab_body,
        grid=(grid,),
        in_specs=[
            pl.BlockSpec((rows, D_FEAT), lambda i: (i, 0)),
            pl.BlockSpec((D_FEAT, D_OUT), lambda i: (0, 0)),
            pl.BlockSpec((D_FEAT, D_OUT), lambda i: (0, 0)),
            pl.BlockSpec((1, D_OUT), lambda i: (0, 0)),
        ],
        out_specs=[
            pl.BlockSpec((rows, D_OUT), lambda i: (i, 0)),
            pl.BlockSpec((rows, D_OUT), lambda i: (i, 0)),
        ],
        out_shape=[
            jax.ShapeDtypeStruct((N_NODES, D_OUT), jnp.float32),
            jax.ShapeDtypeStruct((N_NODES, D_OUT), jnp.float32),
        ],
    )(x, w1, w2, b2)


def _e_body(ea_ref, w3_ref, e_ref):
    e_ref[...] = jnp.dot(ea_ref[...], w3_ref[...],
                         preferred_element_type=jnp.float32)


def _compute_e(ea, w3):
    grid = 125
    rows = N_EDGES // grid
    return pl.pallas_call(
        _e_body,
        grid=(grid,),
        in_specs=[
            pl.BlockSpec((rows, D_EDGE), lambda i: (i, 0)),
            pl.BlockSpec((D_EDGE, D_OUT), lambda i: (0, 0)),
        ],
        out_specs=pl.BlockSpec((rows, D_OUT), lambda i: (i, 0)),
        out_shape=jax.ShapeDtypeStruct((N_EDGES, D_OUT), jnp.float32),
    )(ea, w3)


def _sc_body(a_hbm, b_hbm, e_hbm, dst_hbm, src_hbm, out_hbm,
             buf_a, buf_b, buf_e, dst_g, dst_s, src_g, acc,
             sem_a, sem_b, sem_e, sem_gi, sem_si):
    c = lax.axis_index("c")
    s = lax.axis_index("s")
    wid = s * NC + c

    ba0 = buf_a[0]

    def zero_buf(e, carry):
        for j in range(8):
            ba0[e, pl.ds(j * 16, 16)] = jnp.zeros((16,), jnp.float32)
        return carry
    lax.fori_loop(0, C, zero_buf, 0)
    row0 = s * SPLIT
    for k in range(TAIL // C):
        pltpu.sync_copy(ba0, acc.at[pl.ds(row0 + k * C, C)])
    if TAIL % C:
        pltpu.sync_copy(ba0.at[pl.ds(0, TAIL % C)],
                        acc.at[pl.ds(row0 + (TAIL // C) * C, TAIL % C)])

    @pl.when(s < NS - 1)
    def _zero_rest():
        rest = SPLIT - TAIL
        for k in range(rest // C):
            pltpu.sync_copy(ba0, acc.at[pl.ds(row0 + TAIL + k * C, C)])
        if rest % C:
            pltpu.sync_copy(
                ba0.at[pl.ds(0, rest % C)],
                acc.at[pl.ds(row0 + TAIL + (rest // C) * C, rest % C)])

    plsc.subcore_barrier()

    base0 = wid * EPW

    def issue_gidx(i, p):
        base = base0 + i * C
        pltpu.async_copy(dst_hbm.at[pl.ds(base, C)], dst_g[p], sem_gi[p])
        pltpu.async_copy(src_hbm.at[pl.ds(base, C)], src_g[p], sem_gi[p])

    def wait_gidx(p):
        pltpu.make_async_copy(dst_hbm.at[pl.ds(base0, C)], dst_g[p],
                              sem_gi[p]).wait()
        pltpu.make_async_copy(src_hbm.at[pl.ds(base0, C)], src_g[p],
                              sem_gi[p]).wait()

    def issue_sidx(i, p):
        base = base0 + i * C
        pltpu.async_copy(dst_hbm.at[pl.ds(base, C)], dst_s[p].at[0],
                         sem_si[p])

    def wait_sidx(p):
        pltpu.make_async_copy(dst_hbm.at[pl.ds(base0, C)], dst_s[p].at[0],
                              sem_si[p]).wait()

    def issue_gathers(i, p):
        base = base0 + i * C
        pltpu.async_copy(a_hbm.at[dst_g[p]], buf_a[p], sem_a[p])
        pltpu.async_copy(b_hbm.at[src_g[p]], buf_b[p], sem_b[p])
        pltpu.async_copy(e_hbm.at[pl.ds(base, C)], buf_e[p], sem_e[p])

    def wait_gathers(p):
        pltpu.make_async_copy(a_hbm.at[dst_g[p]], buf_a[p], sem_a[p]).wait()
        pltpu.make_async_copy(b_hbm.at[src_g[p]], buf_b[p], sem_b[p]).wait()
        pltpu.make_async_copy(e_hbm.at[pl.ds(base0, C)], buf_e[p],
                              sem_e[p]).wait()

    def compute_scatter(p):
        ba, bb, be = buf_a[p], buf_b[p], buf_e[p]

        def compute(e, inner):
            for j in range(8):
                sl = pl.ds(j * 16, 16)
                ba[e, sl] = jnp.maximum(ba[e, sl] + bb[e, sl] + be[e, sl],
                                        0.0)
            return inner
        lax.fori_loop(0, C, compute, 0)
        wait_sidx(p)
        pltpu.sync_copy(ba, acc.at[dst_s[p].at[0]], add=True)

    pltpu.sync_copy(dst_hbm.at[pl.ds(base0, C)], dst_g[0])
    pltpu.sync_copy(src_hbm.at[pl.ds(base0, C)], src_g[0])
    issue_gidx(1, 1)
    issue_sidx(0, 0)
    issue_sidx(1, 1)
    issue_gathers(0, 0)

    def step(i, p):
        q = 1 - p
        wait_gidx(q)
        issue_gathers(i + 1, q)
        wait_gathers(p)
        issue_gidx(jnp.minimum(i + 2, NCH - 1), p)
        compute_scatter(p)
        issue_sidx(jnp.minimum(i + 2, NCH - 1), p)

    def pair(k, carry):
        step(2 * k, 0)
        step(2 * k + 1, 1)
        return carry

    lax.fori_loop(0, (NCH - 2) // 2, pair, 0)
    step(NCH - 2, 0)
    wait_gathers(1)
    compute_scatter(1)
    wait_gidx(0)
    wait_sidx(0)
    plsc.subcore_barrier()
    pltpu.sync_copy(acc.at[pl.ds(row0, TAIL)],
                    out_hbm.at[c, pl.ds(row0, TAIL)])

    @pl.when(s < NS - 1)
    def _copy_rest():
        pltpu.sync_copy(acc.at[pl.ds(row0 + TAIL, SPLIT - TAIL)],
                        out_hbm.at[c, pl.ds(row0 + TAIL, SPLIT - TAIL)])


_sc_edge = functools.partial(
    pl.kernel,
    mesh=plsc.VectorSubcoreMesh(core_axis_name="c", subcore_axis_name="s",
                                num_cores=NC, num_subcores=NS),
    out_type=jax.ShapeDtypeStruct((NC, N_NODES, D_OUT), jnp.float32),
    scratch_types=[
        [pltpu.VMEM((C, D_OUT), jnp.float32)] * 2,
        [pltpu.VMEM((C, D_OUT), jnp.float32)] * 2,
        [pltpu.VMEM((C, D_OUT), jnp.float32)] * 2,
        [pltpu.VMEM((C,), jnp.int32)] * 2,
        [pltpu.VMEM((1, C), jnp.int32)] * 2,
        [pltpu.VMEM((C,), jnp.int32)] * 2,
        pltpu.VMEM_SHARED((N_NODES, D_OUT), jnp.float32),
        [pltpu.SemaphoreType.DMA] * 2,
        [pltpu.SemaphoreType.DMA] * 2,
        [pltpu.SemaphoreType.DMA] * 2,
        [pltpu.SemaphoreType.DMA] * 2,
        [pltpu.SemaphoreType.DMA] * 2,
    ],
)(_sc_body)


def _add_body(p_ref, o_ref):
    o_ref[...] = p_ref[0] + p_ref[1]


def _add_partials(p):
    grid = 10
    rows = N_NODES // grid
    return pl.pallas_call(
        _add_body,
        grid=(grid,),
        in_specs=[pl.BlockSpec((NC, rows, D_OUT), lambda i: (0, i, 0))],
        out_specs=pl.BlockSpec((rows, D_OUT), lambda i: (i, 0)),
        out_shape=jax.ShapeDtypeStruct((N_NODES, D_OUT), jnp.float32),
    )(p)


def kernel(Adjacency, node_features, edge_attributes, W, b):
    src = Adjacency[0].astype(jnp.int32)
    dst = Adjacency[1].astype(jnp.int32)
    w1 = W[:D_FEAT]
    w2 = W[D_FEAT:2 * D_FEAT]
    w3 = W[2 * D_FEAT:]
    b2 = b.reshape(1, D_OUT)
    a_tab, b_tab = _compute_ab(node_features, w1, w2, b2)
    e_tab = _compute_e(edge_attributes, w3)
    partials = _sc_edge(a_tab, b_tab, e_tab, dst, src)
    return _add_partials(partials)
